# baseline (speedup 1.0000x reference)
import functools

import jax
import jax.numpy as jnp
from jax.experimental import pallas as pl
from jax.experimental.pallas import tpu as pltpu

_INF = float("inf")
_BIG = 2**31 - 1
_EXH = 3.0e38


def _dot_bf16(qx, qy, kx, ky):
    qxb = qx.astype(jnp.bfloat16).astype(jnp.float32)
    qyb = qy.astype(jnp.bfloat16).astype(jnp.float32)
    kxb = kx.astype(jnp.bfloat16).astype(jnp.float32)
    kyb = ky.astype(jnp.bfloat16).astype(jnp.float32)
    return qxb * kxb + qyb * kyb


def _dst_body(K, radius, q_ref, q2_ref, kx_ref, ky_ref, k2_ref,
              dist_ref, idx_ref, d2_ref,
              l0_ref, l1_ref, l2_ref, l3_ref,
              a0_ref, a1_ref, a2_ref, a3_ref, win_ref):
    B = q_ref.shape[0]
    C = kx_ref.shape[1]
    L = kx_ref.shape[2]
    r2 = radius * radius
    q = q_ref[...]
    qx = q[:, 0:1].reshape(B, 1, 1)
    qy = q[:, 1:2].reshape(B, 1, 1)
    q2 = q2_ref[...].reshape(B, 1, 1)

    liota3 = jax.lax.broadcasted_iota(jnp.int32, (B, C, L), 2)

    dot = _dot_bf16(qx, qy, kx_ref[...], ky_ref[...])
    d2 = jnp.maximum((q2 + k2_ref[...]) - 2.0 * dot, 0.0)
    d2 = jnp.where(d2 <= r2, d2, _INF)
    d2_ref[...] = d2

    lvl_refs = (l0_ref, l1_ref, l2_ref, l3_ref)
    arg_refs = (a0_ref, a1_ref, a2_ref, a3_ref)
    NL = 4
    cur = d2
    for j in range(NL):
        lv = jnp.min(cur, axis=2)
        la = jnp.min(jnp.where(cur == lv[:, :, None], liota3, _BIG), axis=2)
        lvl_refs[j][...] = lv
        arg_refs[j][...] = la
        if j < NL - 1:
            cur = jnp.where(liota3 == la[:, :, None], _INF, cur)
    win_ref[...] = jnp.zeros((B, C), jnp.int32)

    ciota = jax.lax.broadcasted_iota(jnp.int32, (B, C), 1)
    biota = jax.lax.broadcasted_iota(jnp.int32, (B, C), 0)
    kiota = jax.lax.broadcasted_iota(jnp.int32, (B, K), 1)
    liota2 = jax.lax.broadcasted_iota(jnp.int32, (1, L), 1)
    exh = jnp.float32(_EXH)

    def minpos(lvl0):
        m = jnp.min(lvl0, axis=1, keepdims=True)
        cpos = jnp.min(jnp.where(lvl0 == m, ciota, _BIG), axis=1)
        return m, cpos

    def rescue_cond(carry):
        return carry

    def rescue(carry):
        l0 = l0_ref[...]
        for b in range(B):
            ce = jnp.min(jnp.where(l0[b : b + 1, :] == exh,
                                   ciota[b : b + 1, :], _BIG), axis=1)
            need = ce[0] != _BIG
            c = jnp.minimum(ce[0], C - 1)
            chunk = d2_ref[b : b + 1, pl.ds(c, 1), :].reshape(1, L)
            w = jnp.sum(jnp.where(ciota[b : b + 1, :] == c,
                                  win_ref[b : b + 1, :], 0))

            def strip(i, ch):
                mm = jnp.min(ch, axis=1, keepdims=True)
                pp = jnp.min(jnp.where(ch == mm, liota2, _BIG), axis=1,
                             keepdims=True)
                return jnp.where(liota2 == pp, _INF, ch)

            chunk = jax.lax.fori_loop(0, w, strip, chunk)
            sel2 = (ciota == c) & (biota == b) & need
            ch = chunk
            for j in range(NL):
                mm = jnp.min(ch, axis=1, keepdims=True)
                pp = jnp.min(jnp.where(ch == mm, liota2, _BIG), axis=1,
                             keepdims=True)
                lvl_refs[j][...] = jnp.where(sel2, mm[0, 0], lvl_refs[j][...])
                arg_refs[j][...] = jnp.where(sel2, pp[0, 0], arg_refs[j][...])
                if j < NL - 1:
                    ch = jnp.where(liota2 == pp, _INF, ch)
        return jnp.any(l0_ref[...] == exh)

    def step(t, carry):
        dist_acc, idx_acc = carry
        jax.lax.while_loop(rescue_cond, rescue,
                           jnp.any(l0_ref[...] == exh))
        m, cpos = minpos(l0_ref[...])
        hit = ciota == cpos.reshape(B, 1)
        lpos = jnp.sum(jnp.where(hit, a0_ref[...], 0), axis=1, keepdims=True)
        gpos = cpos.reshape(B, 1) * L + lpos
        validv = m <= r2
        dval = jnp.where(validv, m, 0.0)
        ival = jnp.where(validv, gpos, -1)
        sel = kiota == t
        dist_acc = jnp.where(sel, dval, dist_acc)
        idx_acc = jnp.where(sel, ival, idx_acc)
        lvls = [r[...] for r in lvl_refs]
        for j in range(NL - 1):
            lvl_refs[j][...] = jnp.where(hit, lvls[j + 1], lvls[j])
        lvl_refs[NL - 1][...] = jnp.where(hit, exh, lvls[NL - 1])
        args = [r[...] for r in arg_refs]
        for j in range(NL - 1):
            arg_refs[j][...] = jnp.where(hit, args[j + 1], args[j])
        win_ref[...] = win_ref[...] + hit.astype(jnp.int32)
        return dist_acc, idx_acc

    dist_acc = jnp.zeros((B, K), jnp.float32)
    idx_acc = jnp.full((B, K), -1, jnp.int32)
    dist_acc, idx_acc = jax.lax.fori_loop(0, K, step, (dist_acc, idx_acc))
    dist_ref[...] = dist_acc
    idx_ref[...] = idx_acc


def _dst_query(q, q2, kx3, ky3, k23, K, radius, q_block):
    Q = q.shape[0]
    C, L = kx3.shape[1], kx3.shape[2]
    grid = (Q // q_block,)
    body = functools.partial(_dst_body, K, radius)
    return pl.pallas_call(
        body,
        grid=grid,
        in_specs=[
            pl.BlockSpec((q_block, 2), lambda i: (i, 0)),
            pl.BlockSpec((q_block, 1), lambda i: (i, 0)),
            pl.BlockSpec((1, C, L), lambda i: (0, 0, 0)),
            pl.BlockSpec((1, C, L), lambda i: (0, 0, 0)),
            pl.BlockSpec((1, C, L), lambda i: (0, 0, 0)),
        ],
        out_specs=[
            pl.BlockSpec((q_block, K), lambda i: (i, 0)),
            pl.BlockSpec((q_block, K), lambda i: (i, 0)),
        ],
        out_shape=[
            jax.ShapeDtypeStruct((Q, K), jnp.float32),
            jax.ShapeDtypeStruct((Q, K), jnp.int32),
        ],
        scratch_shapes=(
            [pltpu.VMEM((q_block, C, L), jnp.float32)]
            + [pltpu.VMEM((q_block, C), jnp.float32) for _ in range(4)]
            + [pltpu.VMEM((q_block, C), jnp.int32) for _ in range(5)]
        ),
    )(q, q2, kx3, ky3, k23)


def _src_body(K, radius, q_ref, q2_ref, kx_ref, ky_ref, k2_ref,
              dist_ref, idx_ref, d2_ref):
    B = q_ref.shape[0]
    N = kx_ref.shape[1]
    q = q_ref[...]
    qx = q[:, 0:1]
    qy = q[:, 1:2]
    dot = _dot_bf16(qx, qy, kx_ref[...], ky_ref[...])
    d2 = jnp.maximum((q2_ref[...] + k2_ref[...]) - 2.0 * dot, 0.0)
    r2 = radius * radius
    d2_ref[...] = jnp.where(d2 <= r2, d2, _INF)

    iota = jax.lax.broadcasted_iota(jnp.int32, (1, N), 1)
    kiota = jax.lax.broadcasted_iota(jnp.int32, (B, K), 1)

    def step(t, carry):
        dist_acc, idx_acc = carry
        d2c = d2_ref[...]
        m = jnp.min(d2c, axis=1, keepdims=True)
        pos = jnp.min(jnp.where(d2c == m, iota, _BIG), axis=1, keepdims=True)
        valid = m <= r2
        sel = kiota == t
        dist_acc = jnp.where(sel, jnp.where(valid, m, 0.0), dist_acc)
        idx_acc = jnp.where(sel, jnp.where(valid, pos, -1), idx_acc)
        d2_ref[...] = jnp.where(iota == pos, _INF, d2c)
        return dist_acc, idx_acc

    dist_acc = jnp.zeros((B, K), jnp.float32)
    idx_acc = jnp.zeros((B, K), jnp.int32)
    dist_acc, idx_acc = jax.lax.fori_loop(0, K, step, (dist_acc, idx_acc))
    dist_ref[...] = dist_acc
    idx_ref[...] = idx_acc


def _src_query(q, q2, kx, ky, k2, K, radius, q_block):
    Q = q.shape[0]
    N = kx.shape[1]
    grid = (Q // q_block,)
    body = functools.partial(_src_body, K, radius)
    return pl.pallas_call(
        body,
        grid=grid,
        in_specs=[
            pl.BlockSpec((q_block, 2), lambda i: (i, 0)),
            pl.BlockSpec((q_block, 1), lambda i: (i, 0)),
            pl.BlockSpec((1, N), lambda i: (0, 0)),
            pl.BlockSpec((1, N), lambda i: (0, 0)),
            pl.BlockSpec((1, N), lambda i: (0, 0)),
        ],
        out_specs=[
            pl.BlockSpec((q_block, K), lambda i: (i, 0)),
            pl.BlockSpec((q_block, K), lambda i: (i, 0)),
        ],
        out_shape=[
            jax.ShapeDtypeStruct((Q, K), jnp.float32),
            jax.ShapeDtypeStruct((Q, K), jnp.int32),
        ],
        scratch_shapes=[pltpu.VMEM((q_block, N), jnp.float32)],
    )(q, q2, kx, ky, k2)


def kernel(queries, keys):
    q2c = jnp.sum(queries * queries, axis=-1, keepdims=True)
    k2r = jnp.sum(keys * keys, axis=-1, keepdims=True).T
    kx3 = keys[:, 0].reshape(1, 512, 128)
    ky3 = keys[:, 1].reshape(1, 512, 128)
    k23 = k2r.reshape(1, 512, 128)
    qx = queries[:, 0].reshape(1, -1)
    qy = queries[:, 1].reshape(1, -1)
    q2r = q2c.T
    dists_dst, idx_dst = _dst_query(queries, q2c, kx3, ky3, k23, 64, 34.0, 16)
    dists_src, idx_src = _src_query(queries, q2c, qx, qy, q2r, 8, 10.0, 64)
    return dists_dst, idx_dst, dists_src, idx_src


# speedup vs baseline: 2.1465x; 2.1465x over previous
import functools

import jax
import jax.numpy as jnp
from jax.experimental import pallas as pl
from jax.experimental.pallas import tpu as pltpu

_INF = float("inf")
_BIG = 2**31 - 1
_EXH = 3.0e38


def _dot_bf16(qx, qy, kx, ky):
    qxb = qx.astype(jnp.bfloat16).astype(jnp.float32)
    qyb = qy.astype(jnp.bfloat16).astype(jnp.float32)
    kxb = kx.astype(jnp.bfloat16).astype(jnp.float32)
    kyb = ky.astype(jnp.bfloat16).astype(jnp.float32)
    return qxb * kxb + qyb * kyb


def _dst_body(K, radius, q_ref, q2_ref, kx_ref, ky_ref, k2_ref,
              dist_ref, idx_ref, d2_ref,
              l0_ref, l1_ref, l2_ref, l3_ref, l4_ref, l5_ref,
              a0_ref, a1_ref, a2_ref, a3_ref, a4_ref, a5_ref, win_ref):
    B = q_ref.shape[0]
    C = kx_ref.shape[1]
    L = kx_ref.shape[2]
    r2 = radius * radius
    q = q_ref[...]
    qx = q[:, 0:1].reshape(B, 1, 1)
    qy = q[:, 1:2].reshape(B, 1, 1)
    q2 = q2_ref[...].reshape(B, 1, 1)

    liota3 = jax.lax.broadcasted_iota(jnp.int32, (B, C, L), 2)

    dot = _dot_bf16(qx, qy, kx_ref[...], ky_ref[...])
    d2 = jnp.maximum((q2 + k2_ref[...]) - 2.0 * dot, 0.0)
    d2 = jnp.where(d2 <= r2, d2, _INF)
    d2_ref[...] = d2

    lvl_refs = (l0_ref, l1_ref, l2_ref, l3_ref, l4_ref, l5_ref)
    arg_refs = (a0_ref, a1_ref, a2_ref, a3_ref, a4_ref, a5_ref)
    NL = 6
    cur = d2
    for j in range(NL):
        lv = jnp.min(cur, axis=2)
        la = jnp.min(jnp.where(cur == lv[:, :, None], liota3, _BIG), axis=2)
        lvl_refs[j][...] = lv
        arg_refs[j][...] = la
        if j < NL - 1:
            cur = jnp.where(liota3 == la[:, :, None], _INF, cur)
    win_ref[...] = jnp.zeros((B, C), jnp.int32)

    ciota = jax.lax.broadcasted_iota(jnp.int32, (B, C), 1)
    biota = jax.lax.broadcasted_iota(jnp.int32, (B, C), 0)
    kiota = jax.lax.broadcasted_iota(jnp.int32, (B, K), 1)
    liota2 = jax.lax.broadcasted_iota(jnp.int32, (1, L), 1)
    exh = jnp.float32(_EXH)

    def minpos(lvl0):
        m = jnp.min(lvl0, axis=1, keepdims=True)
        cpos = jnp.min(jnp.where(lvl0 == m, ciota, _BIG), axis=1)
        return m, cpos

    def rescue_cond(carry):
        return carry

    def rescue(carry):
        l0 = l0_ref[...]
        for b in range(B):
            ce = jnp.min(jnp.where(l0[b : b + 1, :] == exh,
                                   ciota[b : b + 1, :], _BIG), axis=1)
            need = ce[0] != _BIG
            c = jnp.minimum(ce[0], C - 1)
            chunk = d2_ref[b : b + 1, pl.ds(c, 1), :].reshape(1, L)
            w = jnp.sum(jnp.where(ciota[b : b + 1, :] == c,
                                  win_ref[b : b + 1, :], 0))

            def strip(i, ch):
                mm = jnp.min(ch, axis=1, keepdims=True)
                pp = jnp.min(jnp.where(ch == mm, liota2, _BIG), axis=1,
                             keepdims=True)
                return jnp.where(liota2 == pp, _INF, ch)

            chunk = jax.lax.fori_loop(0, w, strip, chunk)
            sel2 = (ciota == c) & (biota == b) & need
            ch = chunk
            for j in range(NL):
                mm = jnp.min(ch, axis=1, keepdims=True)
                pp = jnp.min(jnp.where(ch == mm, liota2, _BIG), axis=1,
                             keepdims=True)
                lvl_refs[j][...] = jnp.where(sel2, mm[0, 0], lvl_refs[j][...])
                arg_refs[j][...] = jnp.where(sel2, pp[0, 0], arg_refs[j][...])
                if j < NL - 1:
                    ch = jnp.where(liota2 == pp, _INF, ch)
        return jnp.any(l0_ref[...] == exh)

    def step(t, carry):
        dist_acc, idx_acc = carry
        jax.lax.while_loop(rescue_cond, rescue,
                           jnp.any(l0_ref[...] == exh))
        m, cpos = minpos(l0_ref[...])
        hit = ciota == cpos.reshape(B, 1)
        lpos = jnp.sum(jnp.where(hit, a0_ref[...], 0), axis=1, keepdims=True)
        gpos = cpos.reshape(B, 1) * L + lpos
        validv = m <= r2
        dval = jnp.where(validv, m, 0.0)
        ival = jnp.where(validv, gpos, -1)
        sel = kiota == t
        dist_acc = jnp.where(sel, dval, dist_acc)
        idx_acc = jnp.where(sel, ival, idx_acc)
        lvls = [r[...] for r in lvl_refs]
        for j in range(NL - 1):
            lvl_refs[j][...] = jnp.where(hit, lvls[j + 1], lvls[j])
        lvl_refs[NL - 1][...] = jnp.where(hit, exh, lvls[NL - 1])
        args = [r[...] for r in arg_refs]
        for j in range(NL - 1):
            arg_refs[j][...] = jnp.where(hit, args[j + 1], args[j])
        win_ref[...] = win_ref[...] + hit.astype(jnp.int32)
        return dist_acc, idx_acc

    dist_acc = jnp.zeros((B, K), jnp.float32)
    idx_acc = jnp.full((B, K), -1, jnp.int32)
    dist_acc, idx_acc = jax.lax.fori_loop(0, K, step, (dist_acc, idx_acc))
    dist_ref[...] = dist_acc
    idx_ref[...] = idx_acc


def _dst_query(q, q2, kx3, ky3, k23, K, radius, q_block):
    Q = q.shape[0]
    C, L = kx3.shape[1], kx3.shape[2]
    grid = (Q // q_block,)
    body = functools.partial(_dst_body, K, radius)
    return pl.pallas_call(
        body,
        grid=grid,
        in_specs=[
            pl.BlockSpec((q_block, 2), lambda i: (i, 0)),
            pl.BlockSpec((q_block, 1), lambda i: (i, 0)),
            pl.BlockSpec((1, C, L), lambda i: (0, 0, 0)),
            pl.BlockSpec((1, C, L), lambda i: (0, 0, 0)),
            pl.BlockSpec((1, C, L), lambda i: (0, 0, 0)),
        ],
        out_specs=[
            pl.BlockSpec((q_block, K), lambda i: (i, 0)),
            pl.BlockSpec((q_block, K), lambda i: (i, 0)),
        ],
        out_shape=[
            jax.ShapeDtypeStruct((Q, K), jnp.float32),
            jax.ShapeDtypeStruct((Q, K), jnp.int32),
        ],
        scratch_shapes=(
            [pltpu.VMEM((q_block, C, L), jnp.float32)]
            + [pltpu.VMEM((q_block, C), jnp.float32) for _ in range(6)]
            + [pltpu.VMEM((q_block, C), jnp.int32) for _ in range(7)]
        ),
    )(q, q2, kx3, ky3, k23)


def _src_body(K, radius, q_ref, q2_ref, kx_ref, ky_ref, k2_ref,
              dist_ref, idx_ref, d2_ref):
    B = q_ref.shape[0]
    N = kx_ref.shape[1]
    q = q_ref[...]
    qx = q[:, 0:1]
    qy = q[:, 1:2]
    dot = _dot_bf16(qx, qy, kx_ref[...], ky_ref[...])
    d2 = jnp.maximum((q2_ref[...] + k2_ref[...]) - 2.0 * dot, 0.0)
    r2 = radius * radius
    d2_ref[...] = jnp.where(d2 <= r2, d2, _INF)

    iota = jax.lax.broadcasted_iota(jnp.int32, (1, N), 1)
    kiota = jax.lax.broadcasted_iota(jnp.int32, (B, K), 1)

    def step(t, carry):
        dist_acc, idx_acc = carry
        d2c = d2_ref[...]
        m = jnp.min(d2c, axis=1, keepdims=True)
        pos = jnp.min(jnp.where(d2c == m, iota, _BIG), axis=1, keepdims=True)
        valid = m <= r2
        sel = kiota == t
        dist_acc = jnp.where(sel, jnp.where(valid, m, 0.0), dist_acc)
        idx_acc = jnp.where(sel, jnp.where(valid, pos, -1), idx_acc)
        d2_ref[...] = jnp.where(iota == pos, _INF, d2c)
        return dist_acc, idx_acc

    dist_acc = jnp.zeros((B, K), jnp.float32)
    idx_acc = jnp.zeros((B, K), jnp.int32)
    dist_acc, idx_acc = jax.lax.fori_loop(0, K, step, (dist_acc, idx_acc))
    dist_ref[...] = dist_acc
    idx_ref[...] = idx_acc


def _src_query(q, q2, kx, ky, k2, K, radius, q_block):
    Q = q.shape[0]
    N = kx.shape[1]
    grid = (Q // q_block,)
    body = functools.partial(_src_body, K, radius)
    return pl.pallas_call(
        body,
        grid=grid,
        in_specs=[
            pl.BlockSpec((q_block, 2), lambda i: (i, 0)),
            pl.BlockSpec((q_block, 1), lambda i: (i, 0)),
            pl.BlockSpec((1, N), lambda i: (0, 0)),
            pl.BlockSpec((1, N), lambda i: (0, 0)),
            pl.BlockSpec((1, N), lambda i: (0, 0)),
        ],
        out_specs=[
            pl.BlockSpec((q_block, K), lambda i: (i, 0)),
            pl.BlockSpec((q_block, K), lambda i: (i, 0)),
        ],
        out_shape=[
            jax.ShapeDtypeStruct((Q, K), jnp.float32),
            jax.ShapeDtypeStruct((Q, K), jnp.int32),
        ],
        scratch_shapes=[pltpu.VMEM((q_block, N), jnp.float32)],
    )(q, q2, kx, ky, k2)


def kernel(queries, keys):
    q2c = jnp.sum(queries * queries, axis=-1, keepdims=True)
    k2r = jnp.sum(keys * keys, axis=-1, keepdims=True).T
    kx3 = keys[:, 0].reshape(1, 512, 128)
    ky3 = keys[:, 1].reshape(1, 512, 128)
    k23 = k2r.reshape(1, 512, 128)
    qx = queries[:, 0].reshape(1, -1)
    qy = queries[:, 1].reshape(1, -1)
    q2r = q2c.T
    dists_dst, idx_dst = _dst_query(queries, q2c, kx3, ky3, k23, 64, 34.0, 16)
    dists_src, idx_src = _src_query(queries, q2c, qx, qy, q2r, 8, 10.0, 64)
    return dists_dst, idx_dst, dists_src, idx_src


# speedup vs baseline: 2.2900x; 1.0669x over previous
import functools

import jax
import jax.numpy as jnp
from jax.experimental import pallas as pl
from jax.experimental.pallas import tpu as pltpu

_INF = float("inf")
_BIG = 2**31 - 1
_EXH = 3.0e38


def _dot_bf16(qx, qy, kx, ky):
    qxb = qx.astype(jnp.bfloat16).astype(jnp.float32)
    qyb = qy.astype(jnp.bfloat16).astype(jnp.float32)
    kxb = kx.astype(jnp.bfloat16).astype(jnp.float32)
    kyb = ky.astype(jnp.bfloat16).astype(jnp.float32)
    return qxb * kxb + qyb * kyb


def _dst_body(K, radius, q_ref, q2_ref, kx_ref, ky_ref, k2_ref,
              dist_ref, idx_ref, d2_ref,
              l0_ref, l1_ref, l2_ref, l3_ref, l4_ref, l5_ref,
              a0_ref, a1_ref, a2_ref, a3_ref, a4_ref, a5_ref, win_ref):
    B = q_ref.shape[0]
    C = kx_ref.shape[1]
    L = kx_ref.shape[2]
    r2 = radius * radius
    q = q_ref[...]
    qx = q[:, 0:1].reshape(B, 1, 1)
    qy = q[:, 1:2].reshape(B, 1, 1)
    q2 = q2_ref[...].reshape(B, 1, 1)

    liota3 = jax.lax.broadcasted_iota(jnp.int32, (B, C, L), 2)

    dot = _dot_bf16(qx, qy, kx_ref[...], ky_ref[...])
    d2 = jnp.maximum((q2 + k2_ref[...]) - 2.0 * dot, 0.0)
    d2 = jnp.where(d2 <= r2, d2, _INF)
    d2_ref[...] = d2

    lvl_refs = (l0_ref, l1_ref, l2_ref, l3_ref, l4_ref, l5_ref)
    arg_refs = (a0_ref, a1_ref, a2_ref, a3_ref, a4_ref, a5_ref)
    NL = 6
    cur = d2
    for j in range(NL):
        lv = jnp.min(cur, axis=2)
        la = jnp.min(jnp.where(cur == lv[:, :, None], liota3, _BIG), axis=2)
        lvl_refs[j][...] = lv
        arg_refs[j][...] = la
        if j < NL - 1:
            cur = jnp.where(liota3 == la[:, :, None], _INF, cur)
    win_ref[...] = jnp.zeros((B, C), jnp.int32)

    ciota = jax.lax.broadcasted_iota(jnp.int32, (B, C), 1)
    biota = jax.lax.broadcasted_iota(jnp.int32, (B, C), 0)
    kiota = jax.lax.broadcasted_iota(jnp.int32, (B, K), 1)
    liota2 = jax.lax.broadcasted_iota(jnp.int32, (1, L), 1)
    exh = jnp.float32(_EXH)

    def minpos(lvl0):
        m = jnp.min(lvl0, axis=1, keepdims=True)
        cpos = jnp.min(jnp.where(lvl0 == m, ciota, _BIG), axis=1)
        return m, cpos

    def rescue_cond(carry):
        return carry

    def rescue(carry):
        l0 = l0_ref[...]
        for b in range(B):
            ce = jnp.min(jnp.where(l0[b : b + 1, :] == exh,
                                   ciota[b : b + 1, :], _BIG), axis=1)
            need = ce[0] != _BIG
            c = jnp.minimum(ce[0], C - 1)
            chunk = d2_ref[b : b + 1, pl.ds(c, 1), :].reshape(1, L)
            w = jnp.sum(jnp.where(ciota[b : b + 1, :] == c,
                                  win_ref[b : b + 1, :], 0))

            def strip(i, ch):
                mm = jnp.min(ch, axis=1, keepdims=True)
                pp = jnp.min(jnp.where(ch == mm, liota2, _BIG), axis=1,
                             keepdims=True)
                return jnp.where(liota2 == pp, _INF, ch)

            chunk = jax.lax.fori_loop(0, w, strip, chunk)
            sel2 = (ciota == c) & (biota == b) & need
            ch = chunk
            for j in range(NL):
                mm = jnp.min(ch, axis=1, keepdims=True)
                pp = jnp.min(jnp.where(ch == mm, liota2, _BIG), axis=1,
                             keepdims=True)
                lvl_refs[j][...] = jnp.where(sel2, mm[0, 0], lvl_refs[j][...])
                arg_refs[j][...] = jnp.where(sel2, pp[0, 0], arg_refs[j][...])
                if j < NL - 1:
                    ch = jnp.where(liota2 == pp, _INF, ch)
        return jnp.any(l0_ref[...] == exh)

    def step(t, carry):
        dist_acc, idx_acc = carry
        jax.lax.while_loop(rescue_cond, rescue,
                           jnp.any(l0_ref[...] == exh))
        m, cpos = minpos(l0_ref[...])
        hit = ciota == cpos.reshape(B, 1)
        lpos = jnp.sum(jnp.where(hit, a0_ref[...], 0), axis=1, keepdims=True)
        gpos = cpos.reshape(B, 1) * L + lpos
        validv = m <= r2
        dval = jnp.where(validv, m, 0.0)
        ival = jnp.where(validv, gpos, -1)
        sel = kiota == t
        dist_acc = jnp.where(sel, dval, dist_acc)
        idx_acc = jnp.where(sel, ival, idx_acc)
        lvls = [r[...] for r in lvl_refs]
        for j in range(NL - 1):
            lvl_refs[j][...] = jnp.where(hit, lvls[j + 1], lvls[j])
        lvl_refs[NL - 1][...] = jnp.where(hit, exh, lvls[NL - 1])
        args = [r[...] for r in arg_refs]
        for j in range(NL - 1):
            arg_refs[j][...] = jnp.where(hit, args[j + 1], args[j])
        win_ref[...] = win_ref[...] + hit.astype(jnp.int32)
        return dist_acc, idx_acc

    dist_acc = jnp.zeros((B, K), jnp.float32)
    idx_acc = jnp.full((B, K), -1, jnp.int32)
    dist_acc, idx_acc = jax.lax.fori_loop(0, K, step, (dist_acc, idx_acc))
    dist_ref[...] = dist_acc
    idx_ref[...] = idx_acc


def _dst_query(q, q2, kx3, ky3, k23, K, radius, q_block):
    Q = q.shape[0]
    C, L = kx3.shape[1], kx3.shape[2]
    grid = (Q // q_block,)
    body = functools.partial(_dst_body, K, radius)
    return pl.pallas_call(
        body,
        grid=grid,
        in_specs=[
            pl.BlockSpec((q_block, 2), lambda i: (i, 0)),
            pl.BlockSpec((q_block, 1), lambda i: (i, 0)),
            pl.BlockSpec((1, C, L), lambda i: (0, 0, 0)),
            pl.BlockSpec((1, C, L), lambda i: (0, 0, 0)),
            pl.BlockSpec((1, C, L), lambda i: (0, 0, 0)),
        ],
        out_specs=[
            pl.BlockSpec((q_block, K), lambda i: (i, 0)),
            pl.BlockSpec((q_block, K), lambda i: (i, 0)),
        ],
        out_shape=[
            jax.ShapeDtypeStruct((Q, K), jnp.float32),
            jax.ShapeDtypeStruct((Q, K), jnp.int32),
        ],
        scratch_shapes=(
            [pltpu.VMEM((q_block, C, L), jnp.float32)]
            + [pltpu.VMEM((q_block, C), jnp.float32) for _ in range(6)]
            + [pltpu.VMEM((q_block, C), jnp.int32) for _ in range(7)]
        ),
    )(q, q2, kx3, ky3, k23)


def _src_body(K, radius, q_ref, q2_ref, kx_ref, ky_ref, k2_ref,
              dist_ref, idx_ref, d2_ref):
    B = q_ref.shape[0]
    N = kx_ref.shape[1]
    q = q_ref[...]
    qx = q[:, 0:1]
    qy = q[:, 1:2]
    dot = _dot_bf16(qx, qy, kx_ref[...], ky_ref[...])
    d2 = jnp.maximum((q2_ref[...] + k2_ref[...]) - 2.0 * dot, 0.0)
    r2 = radius * radius
    d2_ref[...] = jnp.where(d2 <= r2, d2, _INF)

    iota = jax.lax.broadcasted_iota(jnp.int32, (1, N), 1)
    kiota = jax.lax.broadcasted_iota(jnp.int32, (B, K), 1)

    def step(t, carry):
        dist_acc, idx_acc = carry
        d2c = d2_ref[...]
        m = jnp.min(d2c, axis=1, keepdims=True)
        pos = jnp.min(jnp.where(d2c == m, iota, _BIG), axis=1, keepdims=True)
        valid = m <= r2
        sel = kiota == t
        dist_acc = jnp.where(sel, jnp.where(valid, m, 0.0), dist_acc)
        idx_acc = jnp.where(sel, jnp.where(valid, pos, -1), idx_acc)
        d2_ref[...] = jnp.where(iota == pos, _INF, d2c)
        return dist_acc, idx_acc

    dist_acc = jnp.zeros((B, K), jnp.float32)
    idx_acc = jnp.zeros((B, K), jnp.int32)
    dist_acc, idx_acc = jax.lax.fori_loop(0, K, step, (dist_acc, idx_acc))
    dist_ref[...] = dist_acc
    idx_ref[...] = idx_acc


def _src_query(q, q2, kx, ky, k2, K, radius, q_block):
    Q = q.shape[0]
    N = kx.shape[1]
    grid = (Q // q_block,)
    body = functools.partial(_src_body, K, radius)
    return pl.pallas_call(
        body,
        grid=grid,
        in_specs=[
            pl.BlockSpec((q_block, 2), lambda i: (i, 0)),
            pl.BlockSpec((q_block, 1), lambda i: (i, 0)),
            pl.BlockSpec((1, N), lambda i: (0, 0)),
            pl.BlockSpec((1, N), lambda i: (0, 0)),
            pl.BlockSpec((1, N), lambda i: (0, 0)),
        ],
        out_specs=[
            pl.BlockSpec((q_block, K), lambda i: (i, 0)),
            pl.BlockSpec((q_block, K), lambda i: (i, 0)),
        ],
        out_shape=[
            jax.ShapeDtypeStruct((Q, K), jnp.float32),
            jax.ShapeDtypeStruct((Q, K), jnp.int32),
        ],
        scratch_shapes=[pltpu.VMEM((q_block, N), jnp.float32)],
    )(q, q2, kx, ky, k2)


def kernel(queries, keys):
    q2c = jnp.sum(queries * queries, axis=-1, keepdims=True)
    k2r = jnp.sum(keys * keys, axis=-1, keepdims=True).T
    kx3 = keys[:, 0].reshape(1, 512, 128)
    ky3 = keys[:, 1].reshape(1, 512, 128)
    k23 = k2r.reshape(1, 512, 128)
    qx = queries[:, 0].reshape(1, -1)
    qy = queries[:, 1].reshape(1, -1)
    q2r = q2c.T
    dists_dst, idx_dst = _dst_query(queries, q2c, kx3, ky3, k23, 64, 34.0, 32)
    dists_src, idx_src = _src_query(queries, q2c, qx, qy, q2r, 8, 10.0, 64)
    return dists_dst, idx_dst, dists_src, idx_src


# speedup vs baseline: 2.4279x; 1.0602x over previous
import functools

import jax
import jax.numpy as jnp
from jax.experimental import pallas as pl
from jax.experimental.pallas import tpu as pltpu

_INF = float("inf")
_BIG = 2**31 - 1
_EXH = 3.0e38


def _dot_bf16(qx, qy, kx, ky):
    qxb = qx.astype(jnp.bfloat16).astype(jnp.float32)
    qyb = qy.astype(jnp.bfloat16).astype(jnp.float32)
    kxb = kx.astype(jnp.bfloat16).astype(jnp.float32)
    kyb = ky.astype(jnp.bfloat16).astype(jnp.float32)
    return qxb * kxb + qyb * kyb


def _dst_body(K, radius, q_ref, q2_ref, kx_ref, ky_ref, k2_ref,
              dist_ref, idx_ref, d2_ref,
              l0_ref, l1_ref, l2_ref, l3_ref, l4_ref, l5_ref, l6_ref, l7_ref,
              a0_ref, a1_ref, a2_ref, a3_ref, a4_ref, a5_ref, a6_ref, a7_ref,
              win_ref):
    B = q_ref.shape[0]
    C = kx_ref.shape[1]
    L = kx_ref.shape[2]
    r2 = radius * radius
    q = q_ref[...]
    qx = q[:, 0:1].reshape(B, 1, 1)
    qy = q[:, 1:2].reshape(B, 1, 1)
    q2 = q2_ref[...].reshape(B, 1, 1)

    liota3 = jax.lax.broadcasted_iota(jnp.int32, (B, C, L), 2)

    dot = _dot_bf16(qx, qy, kx_ref[...], ky_ref[...])
    d2 = jnp.maximum((q2 + k2_ref[...]) - 2.0 * dot, 0.0)
    d2 = jnp.where(d2 <= r2, d2, _INF)
    d2_ref[...] = d2

    lvl_refs = (l0_ref, l1_ref, l2_ref, l3_ref, l4_ref, l5_ref, l6_ref, l7_ref)
    arg_refs = (a0_ref, a1_ref, a2_ref, a3_ref, a4_ref, a5_ref, a6_ref, a7_ref)
    NL = 8
    cur = d2
    for j in range(NL):
        lv = jnp.min(cur, axis=2)
        la = jnp.min(jnp.where(cur == lv[:, :, None], liota3, _BIG), axis=2)
        lvl_refs[j][...] = lv
        arg_refs[j][...] = la
        if j < NL - 1:
            cur = jnp.where(liota3 == la[:, :, None], _INF, cur)
    win_ref[...] = jnp.zeros((B, C), jnp.int32)

    ciota = jax.lax.broadcasted_iota(jnp.int32, (B, C), 1)
    biota = jax.lax.broadcasted_iota(jnp.int32, (B, C), 0)
    kiota = jax.lax.broadcasted_iota(jnp.int32, (B, K), 1)
    liota2 = jax.lax.broadcasted_iota(jnp.int32, (1, L), 1)
    exh = jnp.float32(_EXH)

    def minpos(lvl0):
        m = jnp.min(lvl0, axis=1, keepdims=True)
        cpos = jnp.min(jnp.where(lvl0 == m, ciota, _BIG), axis=1)
        return m, cpos

    def rescue_cond(carry):
        return carry

    def rescue(carry):
        l0 = l0_ref[...]
        for b in range(B):
            ce = jnp.min(jnp.where(l0[b : b + 1, :] == exh,
                                   ciota[b : b + 1, :], _BIG), axis=1)
            need = ce[0] != _BIG
            c = jnp.minimum(ce[0], C - 1)
            chunk = d2_ref[b : b + 1, pl.ds(c, 1), :].reshape(1, L)
            w = jnp.sum(jnp.where(ciota[b : b + 1, :] == c,
                                  win_ref[b : b + 1, :], 0))

            def strip(i, ch):
                mm = jnp.min(ch, axis=1, keepdims=True)
                pp = jnp.min(jnp.where(ch == mm, liota2, _BIG), axis=1,
                             keepdims=True)
                return jnp.where(liota2 == pp, _INF, ch)

            chunk = jax.lax.fori_loop(0, w, strip, chunk)
            sel2 = (ciota == c) & (biota == b) & need
            ch = chunk
            for j in range(NL):
                mm = jnp.min(ch, axis=1, keepdims=True)
                pp = jnp.min(jnp.where(ch == mm, liota2, _BIG), axis=1,
                             keepdims=True)
                lvl_refs[j][...] = jnp.where(sel2, mm[0, 0], lvl_refs[j][...])
                arg_refs[j][...] = jnp.where(sel2, pp[0, 0], arg_refs[j][...])
                if j < NL - 1:
                    ch = jnp.where(liota2 == pp, _INF, ch)
        return jnp.any(l0_ref[...] == exh)

    def step(t, carry):
        dist_acc, idx_acc = carry
        jax.lax.while_loop(rescue_cond, rescue,
                           jnp.any(l0_ref[...] == exh))
        m, cpos = minpos(l0_ref[...])
        hit = ciota == cpos.reshape(B, 1)
        lpos = jnp.sum(jnp.where(hit, a0_ref[...], 0), axis=1, keepdims=True)
        gpos = cpos.reshape(B, 1) * L + lpos
        validv = m <= r2
        dval = jnp.where(validv, m, 0.0)
        ival = jnp.where(validv, gpos, -1)
        sel = kiota == t
        dist_acc = jnp.where(sel, dval, dist_acc)
        idx_acc = jnp.where(sel, ival, idx_acc)
        lvls = [r[...] for r in lvl_refs]
        for j in range(NL - 1):
            lvl_refs[j][...] = jnp.where(hit, lvls[j + 1], lvls[j])
        lvl_refs[NL - 1][...] = jnp.where(hit, exh, lvls[NL - 1])
        args = [r[...] for r in arg_refs]
        for j in range(NL - 1):
            arg_refs[j][...] = jnp.where(hit, args[j + 1], args[j])
        win_ref[...] = win_ref[...] + hit.astype(jnp.int32)
        return dist_acc, idx_acc

    dist_acc = jnp.zeros((B, K), jnp.float32)
    idx_acc = jnp.full((B, K), -1, jnp.int32)
    dist_acc, idx_acc = jax.lax.fori_loop(0, K, step, (dist_acc, idx_acc))
    dist_ref[...] = dist_acc
    idx_ref[...] = idx_acc


def _dst_query(q, q2, kx3, ky3, k23, K, radius, q_block):
    Q = q.shape[0]
    C, L = kx3.shape[1], kx3.shape[2]
    grid = (Q // q_block,)
    body = functools.partial(_dst_body, K, radius)
    return pl.pallas_call(
        body,
        grid=grid,
        in_specs=[
            pl.BlockSpec((q_block, 2), lambda i: (i, 0)),
            pl.BlockSpec((q_block, 1), lambda i: (i, 0)),
            pl.BlockSpec((1, C, L), lambda i: (0, 0, 0)),
            pl.BlockSpec((1, C, L), lambda i: (0, 0, 0)),
            pl.BlockSpec((1, C, L), lambda i: (0, 0, 0)),
        ],
        out_specs=[
            pl.BlockSpec((q_block, K), lambda i: (i, 0)),
            pl.BlockSpec((q_block, K), lambda i: (i, 0)),
        ],
        out_shape=[
            jax.ShapeDtypeStruct((Q, K), jnp.float32),
            jax.ShapeDtypeStruct((Q, K), jnp.int32),
        ],
        scratch_shapes=(
            [pltpu.VMEM((q_block, C, L), jnp.float32)]
            + [pltpu.VMEM((q_block, C), jnp.float32) for _ in range(8)]
            + [pltpu.VMEM((q_block, C), jnp.int32) for _ in range(9)]
        ),
    )(q, q2, kx3, ky3, k23)


def _src_body(K, radius, q_ref, q2_ref, kx_ref, ky_ref, k2_ref,
              dist_ref, idx_ref, d2_ref):
    B = q_ref.shape[0]
    N = kx_ref.shape[1]
    q = q_ref[...]
    qx = q[:, 0:1]
    qy = q[:, 1:2]
    dot = _dot_bf16(qx, qy, kx_ref[...], ky_ref[...])
    d2 = jnp.maximum((q2_ref[...] + k2_ref[...]) - 2.0 * dot, 0.0)
    r2 = radius * radius
    d2_ref[...] = jnp.where(d2 <= r2, d2, _INF)

    iota = jax.lax.broadcasted_iota(jnp.int32, (1, N), 1)
    kiota = jax.lax.broadcasted_iota(jnp.int32, (B, K), 1)

    def step(t, carry):
        dist_acc, idx_acc = carry
        d2c = d2_ref[...]
        m = jnp.min(d2c, axis=1, keepdims=True)
        pos = jnp.min(jnp.where(d2c == m, iota, _BIG), axis=1, keepdims=True)
        valid = m <= r2
        sel = kiota == t
        dist_acc = jnp.where(sel, jnp.where(valid, m, 0.0), dist_acc)
        idx_acc = jnp.where(sel, jnp.where(valid, pos, -1), idx_acc)
        d2_ref[...] = jnp.where(iota == pos, _INF, d2c)
        return dist_acc, idx_acc

    dist_acc = jnp.zeros((B, K), jnp.float32)
    idx_acc = jnp.zeros((B, K), jnp.int32)
    dist_acc, idx_acc = jax.lax.fori_loop(0, K, step, (dist_acc, idx_acc))
    dist_ref[...] = dist_acc
    idx_ref[...] = idx_acc


def _src_query(q, q2, kx, ky, k2, K, radius, q_block):
    Q = q.shape[0]
    N = kx.shape[1]
    grid = (Q // q_block,)
    body = functools.partial(_src_body, K, radius)
    return pl.pallas_call(
        body,
        grid=grid,
        in_specs=[
            pl.BlockSpec((q_block, 2), lambda i: (i, 0)),
            pl.BlockSpec((q_block, 1), lambda i: (i, 0)),
            pl.BlockSpec((1, N), lambda i: (0, 0)),
            pl.BlockSpec((1, N), lambda i: (0, 0)),
            pl.BlockSpec((1, N), lambda i: (0, 0)),
        ],
        out_specs=[
            pl.BlockSpec((q_block, K), lambda i: (i, 0)),
            pl.BlockSpec((q_block, K), lambda i: (i, 0)),
        ],
        out_shape=[
            jax.ShapeDtypeStruct((Q, K), jnp.float32),
            jax.ShapeDtypeStruct((Q, K), jnp.int32),
        ],
        scratch_shapes=[pltpu.VMEM((q_block, N), jnp.float32)],
    )(q, q2, kx, ky, k2)


def kernel(queries, keys):
    q2c = jnp.sum(queries * queries, axis=-1, keepdims=True)
    k2r = jnp.sum(keys * keys, axis=-1, keepdims=True).T
    kx3 = keys[:, 0].reshape(1, 512, 128)
    ky3 = keys[:, 1].reshape(1, 512, 128)
    k23 = k2r.reshape(1, 512, 128)
    qx = queries[:, 0].reshape(1, -1)
    qy = queries[:, 1].reshape(1, -1)
    q2r = q2c.T
    dists_dst, idx_dst = _dst_query(queries, q2c, kx3, ky3, k23, 64, 34.0, 32)
    dists_src, idx_src = _src_query(queries, q2c, qx, qy, q2r, 8, 10.0, 64)
    return dists_dst, idx_dst, dists_src, idx_src
